# Initial kernel scaffold; baseline (speedup 1.0000x reference)
import jax
import jax.numpy as jnp
from jax.experimental import pallas as pl


def kernel(x, Wg, w_gate, w_up, w_down):
    raise NotImplementedError("write your pallas kernel here")


# speedup vs baseline: 1.7177x; 1.7177x over previous
import jax
import jax.numpy as jnp
from jax.experimental import pallas as pl

T, D, E, K, F = 2048, 1024, 8, 2, 1024


TB = 1024


def _moe_dense_kernel(x_ref, wg_router_ref, wg_ref, wu_ref, wd_ref, out_ref):
    e = pl.program_id(1)
    x = x_ref[...]
    logits = jnp.dot(x, wg_router_ref[...], preferred_element_type=jnp.float32)
    p = jax.nn.softmax(logits, axis=-1)
    i1 = jnp.argmax(p, axis=-1)
    eidx = jax.lax.broadcasted_iota(jnp.int32, p.shape, 1)
    p_masked = jnp.where(eidx == i1[:, None], -jnp.inf, p)
    i2 = jnp.argmax(p_masked, axis=-1)
    p1 = jnp.max(p, axis=-1)
    p2 = jnp.max(p_masked, axis=-1)
    denom = p1 + p2
    w_e = jnp.where(i1 == e, p1 / denom, 0.0) + jnp.where(i2 == e, p2 / denom, 0.0)

    xb = x.astype(jnp.bfloat16)
    wg = wg_ref[0].astype(jnp.bfloat16)
    wu = wu_ref[0].astype(jnp.bfloat16)
    wd = wd_ref[0].astype(jnp.bfloat16)
    g = jnp.dot(xb, wg, preferred_element_type=jnp.float32)
    u = jnp.dot(xb, wu, preferred_element_type=jnp.float32)
    h = (jax.nn.silu(g) * u).astype(jnp.bfloat16)
    y = jnp.dot(h, wd, preferred_element_type=jnp.float32)
    contrib = y * w_e[:, None]

    @pl.when(e == 0)
    def _init():
        out_ref[...] = contrib

    @pl.when(e != 0)
    def _acc():
        out_ref[...] += contrib


def kernel(x, Wg, w_gate, w_up, w_down):
    return pl.pallas_call(
        _moe_dense_kernel,
        grid=(T // TB, E),
        in_specs=[
            pl.BlockSpec((TB, D), lambda t, e: (t, 0)),
            pl.BlockSpec((D, E), lambda t, e: (0, 0)),
            pl.BlockSpec((1, D, F), lambda t, e: (e, 0, 0)),
            pl.BlockSpec((1, D, F), lambda t, e: (e, 0, 0)),
            pl.BlockSpec((1, F, D), lambda t, e: (e, 0, 0)),
        ],
        out_specs=pl.BlockSpec((TB, D), lambda t, e: (t, 0)),
        out_shape=jax.ShapeDtypeStruct((T, D), jnp.float32),
    )(x, Wg, w_gate, w_up, w_down)
